# baseline (speedup 1.0000x reference)
import functools

import jax
import jax.numpy as jnp
from jax import lax
from jax.experimental import pallas as pl
from jax.experimental.pallas import tpu as pltpu
from jax.experimental.pallas import tpu_sc as plsc

VOCAB_ = 100000
HIDDEN_ = 64
BATCH_ = 1024

_info = plsc.get_sparse_core_info()
_NC, _NS = _info.num_cores, _info.num_subcores
_NW = _NC * _NS
_B_PER_W = BATCH_ // _NW

_mesh = plsc.VectorSubcoreMesh(core_axis_name="c", subcore_axis_name="s")


@functools.partial(
    pl.kernel,
    mesh=_mesh,
    out_type=jax.ShapeDtypeStruct((BATCH_, HIDDEN_), jnp.float32),
    scratch_types=[
        pltpu.VMEM((_B_PER_W,), jnp.int32),
        pltpu.VMEM((_B_PER_W, HIDDEN_), jnp.float32),
        pltpu.SemaphoreType.DMA,
    ],
    compiler_params=pltpu.CompilerParams(use_tc_tiling_on_sc=False),
)
def _sc_gather(idx_hbm, table_hbm, out_hbm, idx_v, rows_v, sem):
    wid = lax.axis_index("s") * _NC + lax.axis_index("c")
    base = wid * _B_PER_W
    pltpu.sync_copy(idx_hbm.at[pl.ds(base, _B_PER_W)], idx_v)
    pltpu.async_copy(table_hbm.at[idx_v], rows_v, sem).wait()
    pltpu.sync_copy(rows_v, out_hbm.at[pl.ds(base, _B_PER_W)])


_V_TILE = 4096
_GRID = pl.cdiv(VOCAB_, _V_TILE)
_TAIL = VOCAB_ - (_GRID - 1) * _V_TILE
_NBUF = 3


def _out_copy(buf, o_hbm, sems, step, k, width=_V_TILE):
    return pltpu.make_async_copy(
        buf.at[k, pl.ds(0, width), :],
        o_hbm.at[pl.ds(step * _V_TILE, width), :],
        sems.at[k],
    )


def _proj_body(x_ref, wt_ref, b_ref, o_ref):
    o_ref[...] = lax.dot_general(
        wt_ref[...], x_ref[...],
        dimension_numbers=(((0,), (1,)), ((), ())),
        preferred_element_type=jnp.float32,
    ) + jnp.reshape(b_ref[...], (_V_TILE, 1))


def kernel(input_ids, embed_table, head_w, head_b):
    x = _sc_gather(input_ids.astype(jnp.int32), embed_table)
    logits_t = pl.pallas_call(
        _proj_body,
        grid=(_GRID,),
        in_specs=[
            pl.BlockSpec((BATCH_, HIDDEN_), lambda j: (0, 0)),
            pl.BlockSpec((HIDDEN_, _V_TILE), lambda j: (0, j)),
            pl.BlockSpec((1, 1, _V_TILE), lambda j: (j, 0, 0)),
        ],
        out_specs=pl.BlockSpec((_V_TILE, BATCH_), lambda j: (j, 0)),
        out_shape=jax.ShapeDtypeStruct((VOCAB_, BATCH_), jnp.float32),
    )(x, head_w.T,
      jnp.pad(head_b, (0, _GRID * _V_TILE - VOCAB_)).reshape(_GRID, 1, _V_TILE))
    return logits_t.T


# speedup vs baseline: 1.0026x; 1.0026x over previous
import functools

import jax
import jax.numpy as jnp
from jax import lax
from jax.experimental import pallas as pl
from jax.experimental.pallas import tpu as pltpu
from jax.experimental.pallas import tpu_sc as plsc

VOCAB_ = 100000
HIDDEN_ = 64
BATCH_ = 1024

_info = plsc.get_sparse_core_info()
_NC, _NS = _info.num_cores, _info.num_subcores
_NW = _NC * _NS
_B_PER_W = BATCH_ // _NW

_mesh = plsc.VectorSubcoreMesh(core_axis_name="c", subcore_axis_name="s")


@functools.partial(
    pl.kernel,
    mesh=_mesh,
    out_type=jax.ShapeDtypeStruct((BATCH_, HIDDEN_), jnp.float32),
    scratch_types=[
        pltpu.VMEM((_B_PER_W,), jnp.int32),
        pltpu.VMEM((_B_PER_W, HIDDEN_), jnp.float32),
        pltpu.SemaphoreType.DMA,
    ],
    compiler_params=pltpu.CompilerParams(use_tc_tiling_on_sc=False),
)
def _sc_gather(idx_hbm, table_hbm, out_hbm, idx_v, rows_v, sem):
    wid = lax.axis_index("s") * _NC + lax.axis_index("c")
    base = wid * _B_PER_W
    pltpu.sync_copy(idx_hbm.at[pl.ds(base, _B_PER_W)], idx_v)
    pltpu.async_copy(table_hbm.at[idx_v], rows_v, sem).wait()
    pltpu.sync_copy(rows_v, out_hbm.at[pl.ds(base, _B_PER_W)])


_V_TILE = 4096
_GRID = pl.cdiv(VOCAB_, _V_TILE)


def _proj_body(x_ref, wt_ref, b_ref, o_ref):
    o_ref[...] = lax.dot_general(
        wt_ref[...], x_ref[...],
        dimension_numbers=(((0,), (1,)), ((), ())),
        preferred_element_type=jnp.float32,
    ) + jnp.reshape(b_ref[...], (_V_TILE, 1))


def kernel(input_ids, embed_table, head_w, head_b):
    x = _sc_gather(input_ids.astype(jnp.int32), embed_table)
    logits_t = pl.pallas_call(
        _proj_body,
        grid=(_GRID,),
        in_specs=[
            pl.BlockSpec((BATCH_, HIDDEN_), lambda j: (0, 0)),
            pl.BlockSpec((HIDDEN_, _V_TILE), lambda j: (0, j)),
            pl.BlockSpec((1, 1, _V_TILE), lambda j: (j, 0, 0)),
        ],
        out_specs=pl.BlockSpec((_V_TILE, BATCH_), lambda j: (j, 0)),
        out_shape=jax.ShapeDtypeStruct((VOCAB_, BATCH_), jnp.float32),
    )(x, head_w.T,
      jnp.pad(head_b, (0, _GRID * _V_TILE - VOCAB_)).reshape(_GRID, 1, _V_TILE))
    return logits_t.T
